# baseline (speedup 1.0000x reference)
import functools

import jax
import jax.numpy as jnp
from jax import lax
from jax.experimental import pallas as pl
from jax.experimental.pallas import tpu as pltpu
from jax.experimental.pallas import tpu_sc as plsc

N = 100000
EMB_DIM = 64
D_FEAT = 128
OUT_DIM = EMB_DIM + D_FEAT

R = 400
NB = N // R
NW = 32


def _sc_body(ids_hbm, feats_hbm, table_hbm, out_hbm, idx_v, emb_v, feats_v,
             sem_g, sem_f):
    wid = lax.axis_index("s") * 2 + lax.axis_index("c")
    niter = (NB - wid + NW - 1) // NW

    def body(i, _):
        blk = wid + i * NW
        base = blk * R
        pltpu.sync_copy(ids_hbm.at[pl.ds(base, R)], idx_v)
        gat = pltpu.make_async_copy(table_hbm.at[idx_v], emb_v, sem_g)
        gat.start()
        fcp = pltpu.make_async_copy(feats_hbm.at[pl.ds(base, R)], feats_v,
                                    sem_f)
        fcp.start()
        gat.wait()
        pltpu.sync_copy(emb_v, out_hbm.at[pl.ds(base, R), pl.ds(0, EMB_DIM)])
        fcp.wait()
        pltpu.sync_copy(feats_v,
                        out_hbm.at[pl.ds(base, R), pl.ds(EMB_DIM, D_FEAT)])
        return 0

    lax.fori_loop(0, niter, body, 0)


@jax.jit
def _feature_prep(ids, feats, embed_weight):
    mesh = plsc.VectorSubcoreMesh(core_axis_name="c", subcore_axis_name="s")
    return pl.kernel(
        _sc_body,
        mesh=mesh,
        out_type=jax.ShapeDtypeStruct((N, OUT_DIM), jnp.float32),
        scratch_types=[
            pltpu.VMEM((R,), jnp.int32),
            pltpu.VMEM((R, EMB_DIM), jnp.float32),
            pltpu.VMEM((R, D_FEAT), jnp.float32),
            pltpu.SemaphoreType.DMA,
            pltpu.SemaphoreType.DMA,
        ],
        compiler_params=pltpu.CompilerParams(use_tc_tiling_on_sc=False),
    )(ids, feats, embed_weight)


def kernel(ids, feats, embed_weight):
    return _feature_prep(ids.astype(jnp.int32), feats, embed_weight)
